# Initial kernel scaffold; baseline (speedup 1.0000x reference)
import jax
import jax.numpy as jnp
from jax.experimental import pallas as pl


def kernel(x, edge_index, W1, b1, W2, b2):
    raise NotImplementedError("write your pallas kernel here")


# speedup vs baseline: 16.2192x; 16.2192x over previous
import functools

import jax
import jax.numpy as jnp
from jax import lax
from jax.experimental import pallas as pl
from jax.experimental.pallas import tpu as pltpu
from jax.experimental.pallas import tpu_sc as plsc

N = 10000
E = 320000
F_IN = 128
HID = 16
C = 40
KS = 3

NC = 2
NS = 16
NW = NC * NS
CH = 79
EP = NW * CH * 128
NP = 10112
RPT = NP // NS
NP3 = KS * NP
CH3 = KS * CH
RPT3 = NP3 // NS

_mesh = plsc.VectorSubcoreMesh(core_axis_name="c", subcore_axis_name="s")


def _make_agg(nrows, ch, rpt):
    @functools.partial(
        pl.kernel,
        mesh=_mesh,
        compiler_params=pltpu.CompilerParams(use_tc_tiling_on_sc=False),
        out_type=jax.ShapeDtypeStruct((NC, nrows, HID), jnp.float32),
        scratch_types=[
            pltpu.VMEM((128,), jnp.int32),
            pltpu.VMEM((128,), jnp.int32),
            pltpu.VMEM((128, HID), jnp.float32),
            pltpu.VMEM_SHARED((nrows, HID), jnp.float32),
            pltpu.VMEM_SHARED((nrows, HID), jnp.float32),
            pltpu.SemaphoreType.DMA,
        ],
    )
    def agg(g_hbm, src_hbm, dst_hbm, z_hbm, out_hbm,
            idxs, idxd, rows, table, acc, sem):
        cid = lax.axis_index("c")
        sid = lax.axis_index("s")
        wid = sid * NC + cid
        sl = pl.ds(sid * rpt, rpt)
        pltpu.sync_copy(g_hbm.at[sl], table.at[sl])
        pltpu.sync_copy(z_hbm, acc.at[sl])
        plsc.subcore_barrier()

        def body(j, carry):
            base = (wid * ch + j) * 128
            pltpu.sync_copy(src_hbm.at[pl.ds(base, 128)], idxs)
            pltpu.sync_copy(dst_hbm.at[pl.ds(base, 128)], idxd)
            pltpu.async_copy(table.at[idxs], rows, sem).wait()
            pltpu.sync_copy(rows, acc.at[idxd], add=True)
            return carry

        lax.fori_loop(0, ch, body, 0)
        plsc.subcore_barrier()
        pltpu.sync_copy(acc.at[sl], out_hbm.at[cid, sl])

    return agg


_agg1 = _make_agg(NP, CH, RPT)


@functools.partial(
    pl.kernel,
    mesh=_mesh,
    compiler_params=pltpu.CompilerParams(use_tc_tiling_on_sc=False),
    out_type=jax.ShapeDtypeStruct((NC, NP, HID), jnp.float32),
    scratch_types=[
        pltpu.VMEM((128,), jnp.int32),
        pltpu.VMEM((128, HID), jnp.float32),
        pltpu.VMEM_SHARED((NP, HID), jnp.float32),
    ],
)
def _deg_kernel(dst_hbm, ones_hbm, z_hbm, out_hbm, idxd, rows, acc):
    cid = lax.axis_index("c")
    sid = lax.axis_index("s")
    wid = sid * NC + cid
    pltpu.sync_copy(ones_hbm, rows)
    sl = pl.ds(sid * RPT, RPT)
    pltpu.sync_copy(z_hbm, acc.at[sl])
    plsc.subcore_barrier()

    def body(j, carry):
        base = (wid * CH + j) * 128
        pltpu.sync_copy(dst_hbm.at[pl.ds(base, 128)], idxd)
        pltpu.sync_copy(rows, acc.at[idxd], add=True)
        return carry

    lax.fori_loop(0, CH, body, 0)
    plsc.subcore_barrier()
    pltpu.sync_copy(acc.at[sl], out_hbm.at[cid, sl])


def _tc1_body(x_ref, w1_ref, degp_ref, g1_ref, dinv_ref):
    deg = degp_ref[0, :, 0] + degp_ref[1, :, 0] + 1.0
    dinv = lax.rsqrt(deg)
    h = jnp.dot(x_ref[...], w1_ref[...], preferred_element_type=jnp.float32)
    g1_ref[...] = h * dinv[:, None]
    dinv_ref[...] = dinv[:, None]


_tc1 = pl.pallas_call(
    _tc1_body,
    out_shape=[
        jax.ShapeDtypeStruct((NP, HID), jnp.float32),
        jax.ShapeDtypeStruct((NP, 1), jnp.float32),
    ],
)


def _tc2_body(up_ref, g1_ref, dinv_ref, w2_ref, b1_ref, g2_ref):
    u1 = up_ref[0] + up_ref[1]
    dinv = dinv_ref[...]
    z1 = dinv * (u1 + g1_ref[...]) + b1_ref[...]
    a1 = jnp.maximum(z1, 0.0)
    h2 = jnp.dot(a1, w2_ref[...], preferred_element_type=jnp.float32)
    g2 = h2 * dinv
    rowid = lax.broadcasted_iota(jnp.int32, (NP, 1), 0)
    g2 = jnp.where(rowid < N, g2, 0.0)
    for k in range(KS):
        g2_ref[k] = g2[:, k * HID:(k + 1) * HID]


_tc2 = pl.pallas_call(
    _tc2_body,
    out_shape=jax.ShapeDtypeStruct((KS, NP, HID), jnp.float32),
)


def _tc3_body(up0_ref, up1_ref, up2_ref, g2_ref, dinv_ref, b2_ref, out_ref):
    dinv = dinv_ref[...]
    zs = []
    for k, upk in enumerate((up0_ref, up1_ref, up2_ref)):
        zs.append(dinv * (upk[0] + upk[1] + g2_ref[k]))
    z = jnp.concatenate(zs, axis=1)[:, :C] + b2_ref[...]
    m = jnp.max(z, axis=1, keepdims=True)
    e = jnp.exp(z - m)
    s = jnp.sum(e, axis=1, keepdims=True)
    out_ref[...] = z - m - jnp.log(s)


_BB = NP // 8
_tc3 = pl.pallas_call(
    _tc3_body,
    grid=(8,),
    in_specs=[
        pl.BlockSpec((NC, _BB, HID), lambda i: (0, i, 0)),
        pl.BlockSpec((NC, _BB, HID), lambda i: (0, i, 0)),
        pl.BlockSpec((NC, _BB, HID), lambda i: (0, i, 0)),
        pl.BlockSpec((KS, _BB, HID), lambda i: (0, i, 0)),
        pl.BlockSpec((_BB, 1), lambda i: (i, 0)),
        pl.BlockSpec((1, C), lambda i: (0, 0)),
    ],
    out_specs=pl.BlockSpec((_BB, C), lambda i: (i, 0)),
    out_shape=jax.ShapeDtypeStruct((NP, C), jnp.float32),
)


def kernel(x, edge_index, W1, b1, W2, b2):
    src = edge_index[0]
    dst = edge_index[1]
    pad = jnp.full((EP - E,), N, jnp.int32)
    src3 = jnp.concatenate([src, pad])
    dst3 = jnp.concatenate([dst, pad])
    x_p = jnp.concatenate(
        [x, jnp.zeros((NP - N, F_IN), jnp.float32)], axis=0)
    z1h = jnp.zeros((RPT, HID), jnp.float32)
    w2p = jnp.concatenate(
        [W2, jnp.zeros((HID, KS * HID - C), jnp.float32)], axis=1)
    ones_rows = jnp.ones((128, HID), jnp.float32)

    degp = _deg_kernel(dst3, ones_rows, z1h)
    g1, dinv = _tc1(x_p, W1, degp)
    up1 = _agg1(g1, src3, dst3, z1h)
    g2 = _tc2(up1, g1, dinv, w2p, b1.reshape(1, HID))
    up2 = [_agg1(g2[k], src3, dst3, z1h) for k in range(KS)]
    outp = _tc3(up2[0], up2[1], up2[2], g2, dinv, b2.reshape(1, C))
    return outp[:N]


# speedup vs baseline: 33.5630x; 2.0693x over previous
import functools

import jax
import jax.numpy as jnp
from jax import lax
from jax.experimental import pallas as pl
from jax.experimental.pallas import tpu as pltpu
from jax.experimental.pallas import tpu_sc as plsc

N = 10000
E = 320000
F_IN = 128
HID = 16
C = 40
KS = 3

NC = 2
NS = 16
NW = NC * NS
CH = 79
EP = NW * CH * 128
NP = 10112
RPT = NP // NS
NP3 = KS * NP
CH3 = KS * CH
RPT3 = NP3 // NS

_mesh = plsc.VectorSubcoreMesh(core_axis_name="c", subcore_axis_name="s")


def _make_agg(nrows, ch, rpt):
    @functools.partial(
        pl.kernel,
        mesh=_mesh,
        compiler_params=pltpu.CompilerParams(use_tc_tiling_on_sc=False),
        out_type=jax.ShapeDtypeStruct((NC, nrows, HID), jnp.float32),
        scratch_types=[
            pltpu.VMEM((ch, 128), jnp.int32),
            pltpu.VMEM((ch, 128), jnp.int32),
            pltpu.VMEM((128, HID), jnp.float32),
            pltpu.VMEM((128, HID), jnp.float32),
            pltpu.VMEM_SHARED((nrows, HID), jnp.float32),
            pltpu.VMEM_SHARED((nrows, HID), jnp.float32),
            pltpu.SemaphoreType.DMA,
            pltpu.SemaphoreType.DMA,
            pltpu.SemaphoreType.DMA,
            pltpu.SemaphoreType.DMA,
        ],
    )
    def agg(g_hbm, src_hbm, dst_hbm, z_hbm, out_hbm,
            srcv, dstv, rows0, rows1, table, acc,
            semg0, semg1, sems0, sems1):
        cid = lax.axis_index("c")
        sid = lax.axis_index("s")
        wid = sid * NC + cid
        pltpu.sync_copy(src_hbm.at[wid], srcv)
        pltpu.sync_copy(dst_hbm.at[wid], dstv)
        sl = pl.ds(sid * rpt, rpt)
        pltpu.sync_copy(g_hbm.at[sl], table.at[sl])
        pltpu.sync_copy(z_hbm, acc.at[sl])
        plsc.subcore_barrier()

        rows = (rows0, rows1)
        semg = (semg0, semg1)
        sems = (sems0, sems1)
        pltpu.async_copy(table.at[srcv.at[0]], rows0, semg0)
        pltpu.async_copy(table.at[srcv.at[1]], rows1, semg1)

        def body(i, carry):
            for b in range(2):
                j = i * 2 + b
                cg = pltpu.make_async_copy(table.at[srcv.at[j]],
                                           rows[b], semg[b])
                cg.wait()
                pltpu.async_copy(rows[b], acc.at[dstv.at[j]], sems[b],
                                 add=True)
                @pl.when(j + 2 < ch)
                def _():
                    cs = pltpu.make_async_copy(rows[b],
                                               acc.at[dstv.at[j]], sems[b])
                    cs.wait()
                    pltpu.async_copy(table.at[srcv.at[j + 2]],
                                     rows[b], semg[b])
            return carry

        lax.fori_loop(0, ch // 2, body, 0)
        pltpu.make_async_copy(table.at[srcv.at[ch - 1]],
                              rows0, semg0).wait()
        pltpu.sync_copy(rows0, acc.at[dstv.at[ch - 1]], add=True)
        pltpu.make_async_copy(rows1, acc.at[dstv.at[ch - 2]], sems1).wait()
        plsc.subcore_barrier()
        pltpu.sync_copy(acc.at[sl], out_hbm.at[cid, sl])

    return agg


_agg1 = _make_agg(NP, CH, RPT)


@functools.partial(
    pl.kernel,
    mesh=_mesh,
    compiler_params=pltpu.CompilerParams(use_tc_tiling_on_sc=False),
    out_type=jax.ShapeDtypeStruct((NC, NP, HID), jnp.float32),
    scratch_types=[
        pltpu.VMEM((128,), jnp.int32),
        pltpu.VMEM((128, HID), jnp.float32),
        pltpu.VMEM_SHARED((NP, HID), jnp.float32),
    ],
)
def _deg_kernel(dst_hbm, ones_hbm, z_hbm, out_hbm, idxd, rows, acc):
    cid = lax.axis_index("c")
    sid = lax.axis_index("s")
    wid = sid * NC + cid
    pltpu.sync_copy(ones_hbm, rows)
    sl = pl.ds(sid * RPT, RPT)
    pltpu.sync_copy(z_hbm, acc.at[sl])
    plsc.subcore_barrier()

    def body(j, carry):
        base = (wid * CH + j) * 128
        pltpu.sync_copy(dst_hbm.at[pl.ds(base, 128)], idxd)
        pltpu.sync_copy(rows, acc.at[idxd], add=True)
        return carry

    lax.fori_loop(0, CH, body, 0)
    plsc.subcore_barrier()
    pltpu.sync_copy(acc.at[sl], out_hbm.at[cid, sl])


def _tc1_body(x_ref, w1_ref, degp_ref, g1_ref, dinv_ref):
    deg = degp_ref[0, :, 0] + degp_ref[1, :, 0] + 1.0
    dinv = lax.rsqrt(deg)
    h = jnp.dot(x_ref[...], w1_ref[...], preferred_element_type=jnp.float32)
    g1_ref[...] = h * dinv[:, None]
    dinv_ref[...] = dinv[:, None]


_tc1 = pl.pallas_call(
    _tc1_body,
    out_shape=[
        jax.ShapeDtypeStruct((NP, HID), jnp.float32),
        jax.ShapeDtypeStruct((NP, 1), jnp.float32),
    ],
)


def _tc2_body(up_ref, g1_ref, dinv_ref, w2_ref, b1_ref, g2_ref):
    u1 = up_ref[0] + up_ref[1]
    dinv = dinv_ref[...]
    z1 = dinv * (u1 + g1_ref[...]) + b1_ref[...]
    a1 = jnp.maximum(z1, 0.0)
    h2 = jnp.dot(a1, w2_ref[...], preferred_element_type=jnp.float32)
    g2 = h2 * dinv
    rowid = lax.broadcasted_iota(jnp.int32, (NP, 1), 0)
    g2 = jnp.where(rowid < N, g2, 0.0)
    for k in range(KS):
        g2_ref[k] = g2[:, k * HID:(k + 1) * HID]


_tc2 = pl.pallas_call(
    _tc2_body,
    out_shape=jax.ShapeDtypeStruct((KS, NP, HID), jnp.float32),
)


def _tc3_body(up0_ref, up1_ref, up2_ref, g2_ref, dinv_ref, b2_ref, out_ref):
    dinv = dinv_ref[...]
    zs = []
    for k, upk in enumerate((up0_ref, up1_ref, up2_ref)):
        zs.append(dinv * (upk[0] + upk[1] + g2_ref[k]))
    z = jnp.concatenate(zs, axis=1)[:, :C] + b2_ref[...]
    m = jnp.max(z, axis=1, keepdims=True)
    e = jnp.exp(z - m)
    s = jnp.sum(e, axis=1, keepdims=True)
    out_ref[...] = z - m - jnp.log(s)


_BB = NP // 8
_tc3 = pl.pallas_call(
    _tc3_body,
    grid=(8,),
    in_specs=[
        pl.BlockSpec((NC, _BB, HID), lambda i: (0, i, 0)),
        pl.BlockSpec((NC, _BB, HID), lambda i: (0, i, 0)),
        pl.BlockSpec((NC, _BB, HID), lambda i: (0, i, 0)),
        pl.BlockSpec((KS, _BB, HID), lambda i: (0, i, 0)),
        pl.BlockSpec((_BB, 1), lambda i: (i, 0)),
        pl.BlockSpec((1, C), lambda i: (0, 0)),
    ],
    out_specs=pl.BlockSpec((_BB, C), lambda i: (i, 0)),
    out_shape=jax.ShapeDtypeStruct((NP, C), jnp.float32),
)


def kernel(x, edge_index, W1, b1, W2, b2):
    src = edge_index[0]
    dst = edge_index[1]
    pad = jnp.full((EP - E,), N, jnp.int32)
    src3 = jnp.concatenate([src, pad]).reshape(NW, CH, 128)
    dst3 = jnp.concatenate([dst, pad]).reshape(NW, CH, 128)
    x_p = jnp.concatenate(
        [x, jnp.zeros((NP - N, F_IN), jnp.float32)], axis=0)
    z1h = jnp.zeros((RPT, HID), jnp.float32)
    w2p = jnp.concatenate(
        [W2, jnp.zeros((HID, KS * HID - C), jnp.float32)], axis=1)
    ones_rows = jnp.ones((128, HID), jnp.float32)

    dflat = dst3.reshape(EP)
    degp = _deg_kernel(dflat, ones_rows, z1h)
    g1, dinv = _tc1(x_p, W1, degp)
    up1 = _agg1(g1, src3, dst3, z1h)
    g2 = _tc2(up1, g1, dinv, w2p, b1.reshape(1, HID))
    up2 = [_agg1(g2[k], src3, dst3, z1h) for k in range(KS)]
    outp = _tc3(up2[0], up2[1], up2[2], g2, dinv, b2.reshape(1, C))
    return outp[:N]


# speedup vs baseline: 38.3947x; 1.1440x over previous
import functools

import jax
import jax.numpy as jnp
from jax import lax
from jax.experimental import pallas as pl
from jax.experimental.pallas import tpu as pltpu
from jax.experimental.pallas import tpu_sc as plsc

N = 10000
E = 320000
F_IN = 128
HID = 16
C = 40
KS = 3

NC = 2
NS = 16
NW = NC * NS
CH = 79
EP = NW * CH * 128
NP = 10112
RPT = NP // NS
NP3 = KS * NP
CH3 = KS * CH
RPT3 = NP3 // NS

_mesh = plsc.VectorSubcoreMesh(core_axis_name="c", subcore_axis_name="s")


def _make_agg(nrows, ch, rpt):
    @functools.partial(
        pl.kernel,
        mesh=_mesh,
        compiler_params=pltpu.CompilerParams(use_tc_tiling_on_sc=False),
        out_type=jax.ShapeDtypeStruct((NC, nrows, HID), jnp.float32),
        scratch_types=[
            pltpu.VMEM((ch, 128), jnp.int32),
            pltpu.VMEM((ch, 128), jnp.int32),
            pltpu.VMEM((128, HID), jnp.float32),
            pltpu.VMEM((128, HID), jnp.float32),
            pltpu.VMEM_SHARED((nrows, HID), jnp.float32),
            pltpu.VMEM_SHARED((nrows, HID), jnp.float32),
            pltpu.SemaphoreType.DMA,
            pltpu.SemaphoreType.DMA,
            pltpu.SemaphoreType.DMA,
            pltpu.SemaphoreType.DMA,
        ],
    )
    def agg(g_hbm, src_hbm, dst_hbm, z_hbm, out_hbm,
            srcv, dstv, rows0, rows1, table, acc,
            semg0, semg1, sems0, sems1):
        cid = lax.axis_index("c")
        sid = lax.axis_index("s")
        wid = sid * NC + cid
        pltpu.sync_copy(src_hbm.at[wid], srcv)
        pltpu.sync_copy(dst_hbm.at[wid], dstv)
        sl = pl.ds(sid * rpt, rpt)
        pltpu.sync_copy(g_hbm.at[sl], table.at[sl])
        pltpu.sync_copy(z_hbm, acc.at[sl])
        plsc.subcore_barrier()

        rows = (rows0, rows1)
        semg = (semg0, semg1)
        sems = (sems0, sems1)

        def wait_g(b):
            pltpu.make_async_copy(table.at[srcv.at[0]], rows[b],
                                  semg[b]).wait()

        def wait_s(b):
            pltpu.make_async_copy(rows[b], acc.at[dstv.at[0]],
                                  sems[b]).wait()

        pltpu.async_copy(table.at[srcv.at[0]], rows0, semg0)
        wait_g(0)
        pltpu.async_copy(rows0, acc.at[dstv.at[0]], sems0, add=True)
        pltpu.async_copy(table.at[srcv.at[1]], rows1, semg1)

        def body(i, carry):
            for b, joff in ((1, 1), (0, 2)):
                j = i * 2 + joff
                ob = 1 - b
                wait_s(ob)

                @pl.when(j + 1 < ch)
                def _():
                    pltpu.async_copy(table.at[srcv.at[j + 1]],
                                     rows[ob], semg[ob])

                wait_g(b)
                pltpu.async_copy(rows[b], acc.at[dstv.at[j]], sems[b],
                                 add=True)
            return carry

        lax.fori_loop(0, (ch - 1) // 2, body, 0)
        wait_s(0)
        plsc.subcore_barrier()
        pltpu.sync_copy(acc.at[sl], out_hbm.at[cid, sl])

    return agg


_agg1 = _make_agg(NP, CH, RPT)


@functools.partial(
    pl.kernel,
    mesh=_mesh,
    compiler_params=pltpu.CompilerParams(use_tc_tiling_on_sc=False),
    out_type=jax.ShapeDtypeStruct((NC, NP, HID), jnp.float32),
    scratch_types=[
        pltpu.VMEM((CH, 128), jnp.int32),
        pltpu.VMEM((128, HID), jnp.float32),
        pltpu.VMEM_SHARED((NP, HID), jnp.float32),
        pltpu.SemaphoreType.DMA,
    ],
)
def _deg_kernel(dst_hbm, ones_hbm, z_hbm, out_hbm, dstv, rows, acc, semd):
    cid = lax.axis_index("c")
    sid = lax.axis_index("s")
    wid = sid * NC + cid
    pltpu.sync_copy(dst_hbm.at[wid], dstv)
    pltpu.sync_copy(ones_hbm, rows)
    sl = pl.ds(sid * RPT, RPT)
    pltpu.sync_copy(z_hbm, acc.at[sl])
    plsc.subcore_barrier()

    def body(j, carry):
        pltpu.async_copy(rows, acc.at[dstv.at[j]], semd, add=True)
        return carry

    lax.fori_loop(0, CH, body, 0)

    def drain(j, carry):
        pltpu.make_async_copy(rows, acc.at[dstv.at[0]], semd).wait()
        return carry

    lax.fori_loop(0, CH, drain, 0)
    plsc.subcore_barrier()
    pltpu.sync_copy(acc.at[sl], out_hbm.at[cid, sl])


def _tc1_body(x_ref, w1_ref, degp_ref, g1_ref, dinv_ref):
    deg = degp_ref[0, :, 0] + degp_ref[1, :, 0] + 1.0
    dinv = lax.rsqrt(deg)
    h = jnp.dot(x_ref[...], w1_ref[...], preferred_element_type=jnp.float32)
    g1_ref[...] = h * dinv[:, None]
    dinv_ref[...] = dinv[:, None]


_tc1 = pl.pallas_call(
    _tc1_body,
    out_shape=[
        jax.ShapeDtypeStruct((NP, HID), jnp.float32),
        jax.ShapeDtypeStruct((NP, 1), jnp.float32),
    ],
)


def _tc2_body(up_ref, g1_ref, dinv_ref, w2_ref, b1_ref, g2_ref):
    u1 = up_ref[0] + up_ref[1]
    dinv = dinv_ref[...]
    z1 = dinv * (u1 + g1_ref[...]) + b1_ref[...]
    a1 = jnp.maximum(z1, 0.0)
    h2 = jnp.dot(a1, w2_ref[...], preferred_element_type=jnp.float32)
    g2 = h2 * dinv
    rowid = lax.broadcasted_iota(jnp.int32, (NP, 1), 0)
    g2 = jnp.where(rowid < N, g2, 0.0)
    for k in range(KS):
        g2_ref[k] = g2[:, k * HID:(k + 1) * HID]


_tc2 = pl.pallas_call(
    _tc2_body,
    out_shape=jax.ShapeDtypeStruct((KS, NP, HID), jnp.float32),
)


def _tc3_body(up0_ref, up1_ref, up2_ref, g2_ref, dinv_ref, b2_ref, out_ref):
    dinv = dinv_ref[...]
    zs = []
    for k, upk in enumerate((up0_ref, up1_ref, up2_ref)):
        zs.append(dinv * (upk[0] + upk[1] + g2_ref[k]))
    z = jnp.concatenate(zs, axis=1)[:, :C] + b2_ref[...]
    m = jnp.max(z, axis=1, keepdims=True)
    e = jnp.exp(z - m)
    s = jnp.sum(e, axis=1, keepdims=True)
    out_ref[...] = z - m - jnp.log(s)


_BB = NP // 8
_tc3 = pl.pallas_call(
    _tc3_body,
    grid=(8,),
    in_specs=[
        pl.BlockSpec((NC, _BB, HID), lambda i: (0, i, 0)),
        pl.BlockSpec((NC, _BB, HID), lambda i: (0, i, 0)),
        pl.BlockSpec((NC, _BB, HID), lambda i: (0, i, 0)),
        pl.BlockSpec((KS, _BB, HID), lambda i: (0, i, 0)),
        pl.BlockSpec((_BB, 1), lambda i: (i, 0)),
        pl.BlockSpec((1, C), lambda i: (0, 0)),
    ],
    out_specs=pl.BlockSpec((_BB, C), lambda i: (i, 0)),
    out_shape=jax.ShapeDtypeStruct((NP, C), jnp.float32),
)


def kernel(x, edge_index, W1, b1, W2, b2):
    src = edge_index[0]
    dst = edge_index[1]
    pad = jnp.full((EP - E,), N, jnp.int32)
    src3 = jnp.concatenate([src, pad]).reshape(NW, CH, 128)
    dst3 = jnp.concatenate([dst, pad]).reshape(NW, CH, 128)
    x_p = jnp.concatenate(
        [x, jnp.zeros((NP - N, F_IN), jnp.float32)], axis=0)
    z1h = jnp.zeros((RPT, HID), jnp.float32)
    w2p = jnp.concatenate(
        [W2, jnp.zeros((HID, KS * HID - C), jnp.float32)], axis=1)
    ones_rows = jnp.ones((128, HID), jnp.float32)

    degp = _deg_kernel(dst3, ones_rows, z1h)
    g1, dinv = _tc1(x_p, W1, degp)
    up1 = _agg1(g1, src3, dst3, z1h)
    g2 = _tc2(up1, g1, dinv, w2p, b1.reshape(1, HID))
    up2 = [_agg1(g2[k], src3, dst3, z1h) for k in range(KS)]
    outp = _tc3(up2[0], up2[1], up2[2], g2, dinv, b2.reshape(1, C))
    return outp[:N]


# speedup vs baseline: 40.2874x; 1.0493x over previous
import functools

import jax
import jax.numpy as jnp
from jax import lax
from jax.experimental import pallas as pl
from jax.experimental.pallas import tpu as pltpu
from jax.experimental.pallas import tpu_sc as plsc

N = 10000
E = 320000
F_IN = 128
HID = 16
C = 40
KS = 3

NC = 2
NS = 16
NW = NC * NS
CH = 79
EP = NW * CH * 128
NP = 10112
RPT = NP // NS
NP3 = KS * NP
CH3 = KS * CH
RPT3 = NP3 // NS

_mesh = plsc.VectorSubcoreMesh(core_axis_name="c", subcore_axis_name="s")


def _make_agg(nrows, ch, rpt):
    @functools.partial(
        pl.kernel,
        mesh=_mesh,
        compiler_params=pltpu.CompilerParams(use_tc_tiling_on_sc=False),
        out_type=jax.ShapeDtypeStruct((NC, nrows, HID), jnp.float32),
        scratch_types=[
            pltpu.VMEM((ch, 128), jnp.int32),
            pltpu.VMEM((ch, 128), jnp.int32),
            pltpu.VMEM((128, HID), jnp.float32),
            pltpu.VMEM((128, HID), jnp.float32),
            pltpu.VMEM((128, HID), jnp.float32),
            pltpu.VMEM((128, HID), jnp.float32),
            pltpu.VMEM_SHARED((nrows, HID), jnp.float32),
            pltpu.VMEM_SHARED((nrows, HID), jnp.float32),
            pltpu.SemaphoreType.DMA,
            pltpu.SemaphoreType.DMA,
            pltpu.SemaphoreType.DMA,
            pltpu.SemaphoreType.DMA,
            pltpu.SemaphoreType.DMA,
            pltpu.SemaphoreType.DMA,
            pltpu.SemaphoreType.DMA,
            pltpu.SemaphoreType.DMA,
        ],
    )
    def agg(g_hbm, src_hbm, dst_hbm, z_hbm, out_hbm,
            srcv, dstv, rows0, rows1, rows2, rows3, table, acc,
            semg0, semg1, semg2, semg3, sems0, sems1, sems2, sems3):
        cid = lax.axis_index("c")
        sid = lax.axis_index("s")
        wid = sid * NC + cid
        pltpu.sync_copy(src_hbm.at[wid], srcv)
        pltpu.sync_copy(dst_hbm.at[wid], dstv)
        sl = pl.ds(sid * rpt, rpt)
        pltpu.sync_copy(g_hbm.at[sl], table.at[sl])
        pltpu.sync_copy(z_hbm, acc.at[sl])
        plsc.subcore_barrier()

        rows = (rows0, rows1, rows2, rows3)
        semg = (semg0, semg1, semg2, semg3)
        sems = (sems0, sems1, sems2, sems3)

        def gath(j, b):
            pltpu.async_copy(table.at[srcv.at[j]], rows[b], semg[b])

        def scat(j, b):
            pltpu.async_copy(rows[b], acc.at[dstv.at[j]], sems[b],
                             add=True)

        def wait_g(b):
            pltpu.make_async_copy(table.at[srcv.at[0]], rows[b],
                                  semg[b]).wait()

        def wait_s(b):
            pltpu.make_async_copy(rows[b], acc.at[dstv.at[0]],
                                  sems[b]).wait()

        gath(0, 0)
        gath(1, 1)
        wait_g(0)
        scat(0, 0)
        gath(2, 2)
        wait_g(1)
        scat(1, 1)
        gath(3, 3)

        def body(i, carry):
            for boff in range(4):
                j = i * 4 + 2 + boff
                b = (2 + boff) % 4
                bn = (b + 2) % 4

                @pl.when(j + 2 < ch)
                def _():
                    wait_s(bn)
                    gath(j + 2, bn)

                wait_g(b)
                scat(j, b)
            return carry

        lax.fori_loop(0, (ch - 3) // 4, body, 0)
        wait_g((ch - 1) % 4)
        scat(ch - 1, (ch - 1) % 4)
        for b in range(4):
            wait_s(b)
        plsc.subcore_barrier()
        pltpu.sync_copy(acc.at[sl], out_hbm.at[cid, sl])

    return agg


_agg1 = _make_agg(NP, CH, RPT)


@functools.partial(
    pl.kernel,
    mesh=_mesh,
    compiler_params=pltpu.CompilerParams(use_tc_tiling_on_sc=False),
    out_type=jax.ShapeDtypeStruct((NC, NP, HID), jnp.float32),
    scratch_types=[
        pltpu.VMEM((CH, 128), jnp.int32),
        pltpu.VMEM((128, HID), jnp.float32),
        pltpu.VMEM_SHARED((NP, HID), jnp.float32),
        pltpu.SemaphoreType.DMA,
    ],
)
def _deg_kernel(dst_hbm, ones_hbm, z_hbm, out_hbm, dstv, rows, acc, semd):
    cid = lax.axis_index("c")
    sid = lax.axis_index("s")
    wid = sid * NC + cid
    pltpu.sync_copy(dst_hbm.at[wid], dstv)
    pltpu.sync_copy(ones_hbm, rows)
    sl = pl.ds(sid * RPT, RPT)
    pltpu.sync_copy(z_hbm, acc.at[sl])
    plsc.subcore_barrier()

    def body(j, carry):
        pltpu.async_copy(rows, acc.at[dstv.at[j]], semd, add=True)
        return carry

    lax.fori_loop(0, CH, body, 0)

    def drain(j, carry):
        pltpu.make_async_copy(rows, acc.at[dstv.at[0]], semd).wait()
        return carry

    lax.fori_loop(0, CH, drain, 0)
    plsc.subcore_barrier()
    pltpu.sync_copy(acc.at[sl], out_hbm.at[cid, sl])


def _tc1_body(x_ref, w1_ref, degp_ref, g1_ref, dinv_ref):
    deg = degp_ref[0, :, 0] + degp_ref[1, :, 0] + 1.0
    dinv = lax.rsqrt(deg)
    h = jnp.dot(x_ref[...], w1_ref[...], preferred_element_type=jnp.float32)
    g1_ref[...] = h * dinv[:, None]
    dinv_ref[...] = dinv[:, None]


_tc1 = pl.pallas_call(
    _tc1_body,
    out_shape=[
        jax.ShapeDtypeStruct((NP, HID), jnp.float32),
        jax.ShapeDtypeStruct((NP, 1), jnp.float32),
    ],
)


def _tc2_body(up_ref, g1_ref, dinv_ref, w2_ref, b1_ref, g2_ref):
    u1 = up_ref[0] + up_ref[1]
    dinv = dinv_ref[...]
    z1 = dinv * (u1 + g1_ref[...]) + b1_ref[...]
    a1 = jnp.maximum(z1, 0.0)
    h2 = jnp.dot(a1, w2_ref[...], preferred_element_type=jnp.float32)
    g2 = h2 * dinv
    rowid = lax.broadcasted_iota(jnp.int32, (NP, 1), 0)
    g2 = jnp.where(rowid < N, g2, 0.0)
    for k in range(KS):
        g2_ref[k] = g2[:, k * HID:(k + 1) * HID]


_tc2 = pl.pallas_call(
    _tc2_body,
    out_shape=jax.ShapeDtypeStruct((KS, NP, HID), jnp.float32),
)


def _tc3_body(up0_ref, up1_ref, up2_ref, g2_ref, dinv_ref, b2_ref, out_ref):
    dinv = dinv_ref[...]
    zs = []
    for k, upk in enumerate((up0_ref, up1_ref, up2_ref)):
        zs.append(dinv * (upk[0] + upk[1] + g2_ref[k]))
    z = jnp.concatenate(zs, axis=1)[:, :C] + b2_ref[...]
    m = jnp.max(z, axis=1, keepdims=True)
    e = jnp.exp(z - m)
    s = jnp.sum(e, axis=1, keepdims=True)
    out_ref[...] = z - m - jnp.log(s)


_BB = NP // 8
_tc3 = pl.pallas_call(
    _tc3_body,
    grid=(8,),
    in_specs=[
        pl.BlockSpec((NC, _BB, HID), lambda i: (0, i, 0)),
        pl.BlockSpec((NC, _BB, HID), lambda i: (0, i, 0)),
        pl.BlockSpec((NC, _BB, HID), lambda i: (0, i, 0)),
        pl.BlockSpec((KS, _BB, HID), lambda i: (0, i, 0)),
        pl.BlockSpec((_BB, 1), lambda i: (i, 0)),
        pl.BlockSpec((1, C), lambda i: (0, 0)),
    ],
    out_specs=pl.BlockSpec((_BB, C), lambda i: (i, 0)),
    out_shape=jax.ShapeDtypeStruct((NP, C), jnp.float32),
)


def kernel(x, edge_index, W1, b1, W2, b2):
    src = edge_index[0]
    dst = edge_index[1]
    pad = jnp.full((EP - E,), N, jnp.int32)
    src3 = jnp.concatenate([src, pad]).reshape(NW, CH, 128)
    dst3 = jnp.concatenate([dst, pad]).reshape(NW, CH, 128)
    x_p = jnp.concatenate(
        [x, jnp.zeros((NP - N, F_IN), jnp.float32)], axis=0)
    z1h = jnp.zeros((RPT, HID), jnp.float32)
    w2p = jnp.concatenate(
        [W2, jnp.zeros((HID, KS * HID - C), jnp.float32)], axis=1)
    ones_rows = jnp.ones((128, HID), jnp.float32)

    degp = _deg_kernel(dst3, ones_rows, z1h)
    g1, dinv = _tc1(x_p, W1, degp)
    up1 = _agg1(g1, src3, dst3, z1h)
    g2 = _tc2(up1, g1, dinv, w2p, b1.reshape(1, HID))
    up2 = [_agg1(g2[k], src3, dst3, z1h) for k in range(KS)]
    outp = _tc3(up2[0], up2[1], up2[2], g2, dinv, b2.reshape(1, C))
    return outp[:N]


# speedup vs baseline: 41.9001x; 1.0400x over previous
import functools

import jax
import jax.numpy as jnp
from jax import lax
from jax.experimental import pallas as pl
from jax.experimental.pallas import tpu as pltpu
from jax.experimental.pallas import tpu_sc as plsc

N = 10000
E = 320000
F_IN = 128
HID = 16
C = 40
KS = 3

NC = 2
NS = 16
NW = NC * NS
CH = 79
EP = NW * CH * 128
NP = 10112
RPT = NP // NS
NP3 = KS * NP
CH3 = KS * CH
RPT3 = NP3 // NS

_mesh = plsc.VectorSubcoreMesh(core_axis_name="c", subcore_axis_name="s")


def _make_agg(nslab, ch):
    @functools.partial(
        pl.kernel,
        mesh=_mesh,
        compiler_params=pltpu.CompilerParams(use_tc_tiling_on_sc=False),
        out_type=jax.ShapeDtypeStruct((NC, nslab * NP, HID), jnp.float32),
        scratch_types=[
            pltpu.VMEM((ch, 128), jnp.int32),
            pltpu.VMEM((ch, 128), jnp.int32),
            pltpu.VMEM((128, HID), jnp.float32),
            pltpu.VMEM((128, HID), jnp.float32),
            pltpu.VMEM((128, HID), jnp.float32),
            pltpu.VMEM((128, HID), jnp.float32),
            pltpu.VMEM_SHARED((NP, HID), jnp.float32),
            pltpu.VMEM_SHARED((NP, HID), jnp.float32),
            pltpu.SemaphoreType.DMA,
            pltpu.SemaphoreType.DMA,
            pltpu.SemaphoreType.DMA,
            pltpu.SemaphoreType.DMA,
            pltpu.SemaphoreType.DMA,
            pltpu.SemaphoreType.DMA,
            pltpu.SemaphoreType.DMA,
            pltpu.SemaphoreType.DMA,
        ],
    )
    def agg(g_hbm, src_hbm, dst_hbm, z_hbm, out_hbm,
            srcv, dstv, rows0, rows1, rows2, rows3, table, acc,
            semg0, semg1, semg2, semg3, sems0, sems1, sems2, sems3):
        cid = lax.axis_index("c")
        sid = lax.axis_index("s")
        wid = sid * NC + cid
        pltpu.sync_copy(src_hbm.at[wid], srcv)
        pltpu.sync_copy(dst_hbm.at[wid], dstv)
        sl = pl.ds(sid * RPT, RPT)

        rows = (rows0, rows1, rows2, rows3)
        semg = (semg0, semg1, semg2, semg3)
        sems = (sems0, sems1, sems2, sems3)

        def gath(j, b):
            pltpu.async_copy(table.at[srcv.at[j]], rows[b], semg[b])

        def scat(j, b):
            pltpu.async_copy(rows[b], acc.at[dstv.at[j]], sems[b],
                             add=True)

        def wait_g(b):
            pltpu.make_async_copy(table.at[srcv.at[0]], rows[b],
                                  semg[b]).wait()

        def wait_s(b):
            pltpu.make_async_copy(rows[b], acc.at[dstv.at[0]],
                                  sems[b]).wait()

        def body(i, carry):
            for boff in range(4):
                j = i * 4 + 2 + boff
                b = (2 + boff) % 4
                bn = (b + 2) % 4

                @pl.when(j + 2 < ch)
                def _():
                    wait_s(bn)
                    gath(j + 2, bn)

                wait_g(b)
                scat(j, b)
            return carry

        for k in range(nslab):
            pltpu.sync_copy(
                g_hbm.at[pl.ds(k * NP + sid * RPT, RPT)], table.at[sl])
            pltpu.sync_copy(z_hbm, acc.at[sl])
            plsc.subcore_barrier()
            gath(0, 0)
            gath(1, 1)
            wait_g(0)
            scat(0, 0)
            gath(2, 2)
            wait_g(1)
            scat(1, 1)
            gath(3, 3)
            lax.fori_loop(0, (ch - 3) // 4, body, 0)
            wait_g((ch - 1) % 4)
            scat(ch - 1, (ch - 1) % 4)
            for b in range(4):
                wait_s(b)
            plsc.subcore_barrier()
            pltpu.sync_copy(
                acc.at[sl], out_hbm.at[cid, pl.ds(k * NP + sid * RPT, RPT)])

    return agg


_agg1 = _make_agg(1, CH)
_agg3 = _make_agg(KS, CH)


@functools.partial(
    pl.kernel,
    mesh=_mesh,
    compiler_params=pltpu.CompilerParams(use_tc_tiling_on_sc=False),
    out_type=jax.ShapeDtypeStruct((NC, NP, HID), jnp.float32),
    scratch_types=[
        pltpu.VMEM((CH, 128), jnp.int32),
        pltpu.VMEM((128, HID), jnp.float32),
        pltpu.VMEM_SHARED((NP, HID), jnp.float32),
        pltpu.SemaphoreType.DMA,
    ],
)
def _deg_kernel(dst_hbm, ones_hbm, z_hbm, out_hbm, dstv, rows, acc, semd):
    cid = lax.axis_index("c")
    sid = lax.axis_index("s")
    wid = sid * NC + cid
    pltpu.sync_copy(dst_hbm.at[wid], dstv)
    pltpu.sync_copy(ones_hbm, rows)
    sl = pl.ds(sid * RPT, RPT)
    pltpu.sync_copy(z_hbm, acc.at[sl])
    plsc.subcore_barrier()

    def body(j, carry):
        pltpu.async_copy(rows, acc.at[dstv.at[j]], semd, add=True)
        return carry

    lax.fori_loop(0, CH, body, 0)

    def drain(j, carry):
        pltpu.make_async_copy(rows, acc.at[dstv.at[0]], semd).wait()
        return carry

    lax.fori_loop(0, CH, drain, 0)
    plsc.subcore_barrier()
    pltpu.sync_copy(acc.at[sl], out_hbm.at[cid, sl])


def _tc1_body(x_ref, w1_ref, degp_ref, g1_ref, dinv_ref):
    deg = degp_ref[0, :, 0] + degp_ref[1, :, 0] + 1.0
    dinv = lax.rsqrt(deg)
    h = jnp.dot(x_ref[...], w1_ref[...], preferred_element_type=jnp.float32)
    g1_ref[...] = h * dinv[:, None]
    dinv_ref[...] = dinv[:, None]


_tc1 = pl.pallas_call(
    _tc1_body,
    out_shape=[
        jax.ShapeDtypeStruct((NP, HID), jnp.float32),
        jax.ShapeDtypeStruct((NP, 1), jnp.float32),
    ],
)


def _tc2_body(up_ref, g1_ref, dinv_ref, w2_ref, b1_ref, g2_ref):
    u1 = up_ref[0] + up_ref[1]
    dinv = dinv_ref[...]
    z1 = dinv * (u1 + g1_ref[...]) + b1_ref[...]
    a1 = jnp.maximum(z1, 0.0)
    h2 = jnp.dot(a1, w2_ref[...], preferred_element_type=jnp.float32)
    g2 = h2 * dinv
    rowid = lax.broadcasted_iota(jnp.int32, (NP, 1), 0)
    g2 = jnp.where(rowid < N, g2, 0.0)
    for k in range(KS):
        g2_ref[k] = g2[:, k * HID:(k + 1) * HID]


_tc2 = pl.pallas_call(
    _tc2_body,
    out_shape=jax.ShapeDtypeStruct((KS, NP, HID), jnp.float32),
)


def _tc3_body(up_ref, g2_ref, dinv_ref, b2_ref, out_ref):
    dinv = dinv_ref[...]
    zs = []
    for k in range(KS):
        zs.append(dinv * (up_ref[0, k] + up_ref[1, k] + g2_ref[k]))
    z = jnp.concatenate(zs, axis=1)[:, :C] + b2_ref[...]
    m = jnp.max(z, axis=1, keepdims=True)
    e = jnp.exp(z - m)
    s = jnp.sum(e, axis=1, keepdims=True)
    out_ref[...] = z - m - jnp.log(s)


_BB = NP // 8
_tc3 = pl.pallas_call(
    _tc3_body,
    grid=(8,),
    in_specs=[
        pl.BlockSpec((NC, KS, _BB, HID), lambda i: (0, 0, i, 0)),
        pl.BlockSpec((KS, _BB, HID), lambda i: (0, i, 0)),
        pl.BlockSpec((_BB, 1), lambda i: (i, 0)),
        pl.BlockSpec((1, C), lambda i: (0, 0)),
    ],
    out_specs=pl.BlockSpec((_BB, C), lambda i: (i, 0)),
    out_shape=jax.ShapeDtypeStruct((NP, C), jnp.float32),
)


def kernel(x, edge_index, W1, b1, W2, b2):
    src = edge_index[0]
    dst = edge_index[1]
    pad = jnp.full((EP - E,), N, jnp.int32)
    src3 = jnp.concatenate([src, pad]).reshape(NW, CH, 128)
    dst3 = jnp.concatenate([dst, pad]).reshape(NW, CH, 128)
    x_p = jnp.concatenate(
        [x, jnp.zeros((NP - N, F_IN), jnp.float32)], axis=0)
    z1h = jnp.zeros((RPT, HID), jnp.float32)
    w2p = jnp.concatenate(
        [W2, jnp.zeros((HID, KS * HID - C), jnp.float32)], axis=1)
    ones_rows = jnp.ones((128, HID), jnp.float32)

    degp = _deg_kernel(dst3, ones_rows, z1h)
    g1, dinv = _tc1(x_p, W1, degp)
    up1 = _agg1(g1, src3, dst3, z1h)
    g2 = _tc2(up1, g1, dinv, w2p, b1.reshape(1, HID))
    up2 = _agg3(g2.reshape(KS * NP, HID), src3, dst3, z1h)
    outp = _tc3(up2.reshape(NC, KS, NP, HID), g2, dinv, b2.reshape(1, C))
    return outp[:N]


# speedup vs baseline: 41.9086x; 1.0002x over previous
import functools

import jax
import jax.numpy as jnp
from jax import lax
from jax.experimental import pallas as pl
from jax.experimental.pallas import tpu as pltpu
from jax.experimental.pallas import tpu_sc as plsc

N = 10000
E = 320000
F_IN = 128
HID = 16
C = 40
KS = 3

NC = 2
NS = 16
NW = NC * NS
CH = 79
EP = NW * CH * 128
NP = 10112
RPT = NP // NS
NP3 = KS * NP
CH3 = KS * CH
RPT3 = NP3 // NS

_mesh = plsc.VectorSubcoreMesh(core_axis_name="c", subcore_axis_name="s")


def _make_agg(nslab, ch):
    @functools.partial(
        pl.kernel,
        mesh=_mesh,
        compiler_params=pltpu.CompilerParams(use_tc_tiling_on_sc=False),
        out_type=jax.ShapeDtypeStruct((NC, nslab * NP, HID), jnp.float32),
        scratch_types=[
            pltpu.VMEM((ch, 128), jnp.int32),
            pltpu.VMEM((ch, 128), jnp.int32),
            pltpu.VMEM((128, HID), jnp.float32),
            pltpu.VMEM((128, HID), jnp.float32),
            pltpu.VMEM((128, HID), jnp.float32),
            pltpu.VMEM((128, HID), jnp.float32),
            pltpu.VMEM_SHARED((NP, HID), jnp.float32),
            pltpu.VMEM_SHARED((NP, HID), jnp.float32),
            pltpu.SemaphoreType.DMA,
            pltpu.SemaphoreType.DMA,
            pltpu.SemaphoreType.DMA,
            pltpu.SemaphoreType.DMA,
            pltpu.SemaphoreType.DMA,
            pltpu.SemaphoreType.DMA,
            pltpu.SemaphoreType.DMA,
            pltpu.SemaphoreType.DMA,
        ],
    )
    def agg(g_hbm, src_hbm, dst_hbm, z_hbm, out_hbm,
            srcv, dstv, rows0, rows1, rows2, rows3, table, acc,
            semg0, semg1, semg2, semg3, sems0, sems1, sems2, sems3):
        cid = lax.axis_index("c")
        sid = lax.axis_index("s")
        wid = sid * NC + cid
        pltpu.sync_copy(src_hbm.at[wid], srcv)
        pltpu.sync_copy(dst_hbm.at[wid], dstv)
        sl = pl.ds(sid * RPT, RPT)

        rows = (rows0, rows1, rows2, rows3)
        semg = (semg0, semg1, semg2, semg3)
        sems = (sems0, sems1, sems2, sems3)

        def gath(j, b):
            pltpu.async_copy(table.at[srcv.at[j]], rows[b], semg[b])

        def scat(j, b):
            pltpu.async_copy(rows[b], acc.at[dstv.at[j]], sems[b],
                             add=True)

        def wait_g(b):
            pltpu.make_async_copy(table.at[srcv.at[0]], rows[b],
                                  semg[b]).wait()

        def wait_s(b):
            pltpu.make_async_copy(rows[b], acc.at[dstv.at[0]],
                                  sems[b]).wait()

        def body(i, carry):
            for boff in range(4):
                j = i * 4 + 2 + boff
                b = (2 + boff) % 4
                bn = (b + 2) % 4

                @pl.when(j + 2 < ch)
                def _():
                    wait_s(bn)
                    gath(j + 2, bn)

                wait_g(b)
                scat(j, b)
            return carry

        for k in range(nslab):
            pltpu.sync_copy(
                g_hbm.at[pl.ds(k * NP + sid * RPT, RPT)], table.at[sl])
            pltpu.sync_copy(z_hbm, acc.at[sl])
            plsc.subcore_barrier()
            gath(0, 0)
            gath(1, 1)
            wait_g(0)
            scat(0, 0)
            gath(2, 2)
            wait_g(1)
            scat(1, 1)
            gath(3, 3)
            lax.fori_loop(0, (ch - 3) // 4, body, 0)
            wait_g((ch - 1) % 4)
            scat(ch - 1, (ch - 1) % 4)
            for b in range(4):
                wait_s(b)
            plsc.subcore_barrier()
            pltpu.sync_copy(
                acc.at[sl], out_hbm.at[cid, pl.ds(k * NP + sid * RPT, RPT)])

    return agg


_agg1 = _make_agg(1, CH)
_agg3 = _make_agg(KS, CH)


@functools.partial(
    pl.kernel,
    mesh=_mesh,
    compiler_params=pltpu.CompilerParams(use_tc_tiling_on_sc=False),
    out_type=jax.ShapeDtypeStruct((NC, NP, HID), jnp.float32),
    scratch_types=[
        pltpu.VMEM((CH, 128), jnp.int32),
        pltpu.VMEM((128, HID), jnp.float32),
        pltpu.VMEM_SHARED((NP, HID), jnp.float32),
        pltpu.SemaphoreType.DMA,
    ],
)
def _deg_kernel(dst_hbm, ones_hbm, z_hbm, out_hbm, dstv, rows, acc, semd):
    cid = lax.axis_index("c")
    sid = lax.axis_index("s")
    wid = sid * NC + cid
    pltpu.sync_copy(dst_hbm.at[wid], dstv)
    pltpu.sync_copy(ones_hbm, rows)
    sl = pl.ds(sid * RPT, RPT)
    pltpu.sync_copy(z_hbm, acc.at[sl])
    plsc.subcore_barrier()

    def body(j, carry):
        pltpu.async_copy(rows, acc.at[dstv.at[j]], semd, add=True)
        return carry

    lax.fori_loop(0, CH, body, 0)

    def drain(j, carry):
        pltpu.make_async_copy(rows, acc.at[dstv.at[0]], semd).wait()
        return carry

    lax.fori_loop(0, CH, drain, 0)
    plsc.subcore_barrier()
    pltpu.sync_copy(acc.at[sl], out_hbm.at[cid, sl])


def _tc0_body(x_ref, w1_ref, h_ref):
    h_ref[...] = jnp.dot(x_ref[...], w1_ref[...],
                         preferred_element_type=jnp.float32)


_tc0 = pl.pallas_call(
    _tc0_body,
    out_shape=jax.ShapeDtypeStruct((NP, HID), jnp.float32),
)


def _tc1_body(h_ref, degp_ref, g1_ref, dinv_ref):
    deg = degp_ref[0, :, 0] + degp_ref[1, :, 0] + 1.0
    dinv = lax.rsqrt(deg)
    g1_ref[...] = h_ref[...] * dinv[:, None]
    dinv_ref[...] = dinv[:, None]


_tc1 = pl.pallas_call(
    _tc1_body,
    out_shape=[
        jax.ShapeDtypeStruct((NP, HID), jnp.float32),
        jax.ShapeDtypeStruct((NP, 1), jnp.float32),
    ],
)


def _tc2_body(up_ref, g1_ref, dinv_ref, w2_ref, b1_ref, g2_ref):
    u1 = up_ref[0] + up_ref[1]
    dinv = dinv_ref[...]
    z1 = dinv * (u1 + g1_ref[...]) + b1_ref[...]
    a1 = jnp.maximum(z1, 0.0)
    h2 = jnp.dot(a1, w2_ref[...], preferred_element_type=jnp.float32)
    g2 = h2 * dinv
    rowid = lax.broadcasted_iota(jnp.int32, (NP, 1), 0)
    g2 = jnp.where(rowid < N, g2, 0.0)
    for k in range(KS):
        g2_ref[k] = g2[:, k * HID:(k + 1) * HID]


_tc2 = pl.pallas_call(
    _tc2_body,
    out_shape=jax.ShapeDtypeStruct((KS, NP, HID), jnp.float32),
)


def _tc3_body(up_ref, g2_ref, dinv_ref, b2_ref, out_ref):
    dinv = dinv_ref[...]
    zs = []
    for k in range(KS):
        zs.append(dinv * (up_ref[0, k] + up_ref[1, k] + g2_ref[k]))
    z = jnp.concatenate(zs, axis=1)[:, :C] + b2_ref[...]
    m = jnp.max(z, axis=1, keepdims=True)
    e = jnp.exp(z - m)
    s = jnp.sum(e, axis=1, keepdims=True)
    out_ref[...] = z - m - jnp.log(s)


_BB = NP // 8
_tc3 = pl.pallas_call(
    _tc3_body,
    grid=(8,),
    in_specs=[
        pl.BlockSpec((NC, KS, _BB, HID), lambda i: (0, 0, i, 0)),
        pl.BlockSpec((KS, _BB, HID), lambda i: (0, i, 0)),
        pl.BlockSpec((_BB, 1), lambda i: (i, 0)),
        pl.BlockSpec((1, C), lambda i: (0, 0)),
    ],
    out_specs=pl.BlockSpec((_BB, C), lambda i: (i, 0)),
    out_shape=jax.ShapeDtypeStruct((NP, C), jnp.float32),
)


def kernel(x, edge_index, W1, b1, W2, b2):
    src = edge_index[0]
    dst = edge_index[1]
    pad = jnp.full((EP - E,), N, jnp.int32)
    src3 = jnp.concatenate([src, pad]).reshape(NW, CH, 128)
    dst3 = jnp.concatenate([dst, pad]).reshape(NW, CH, 128)
    x_p = jnp.concatenate(
        [x, jnp.zeros((NP - N, F_IN), jnp.float32)], axis=0)
    z1h = jnp.zeros((RPT, HID), jnp.float32)
    w2p = jnp.concatenate(
        [W2, jnp.zeros((HID, KS * HID - C), jnp.float32)], axis=1)
    ones_rows = jnp.ones((128, HID), jnp.float32)

    h1 = _tc0(x_p, W1)
    degp = _deg_kernel(dst3, ones_rows, z1h)
    g1, dinv = _tc1(h1, degp)
    up1 = _agg1(g1, src3, dst3, z1h)
    g2 = _tc2(up1, g1, dinv, w2p, b1.reshape(1, HID))
    up2 = _agg3(g2.reshape(KS * NP, HID), src3, dst3, z1h)
    outp = _tc3(up2.reshape(NC, KS, NP, HID), g2, dinv, b2.reshape(1, C))
    return outp[:N]
